# baseline (speedup 1.0000x reference)
import jax
import jax.numpy as jnp
from jax.experimental import pallas as pl

_B, _XC, _C, _K, _HW = 8, 3, 64, 1024, 224
_H1 = 112
_H2 = 56
_F32 = jnp.float32
_VQ_CHUNKS = 4

_TCONV_TAPS = {0: ((1, 1), (0, 3)), 1: ((2, 0), (1, 2))}
_C1_TAPS = [(p, q, s, t)
            for p in range(2) for q in range(2)
            for s in range(2) for t in range(2)]


def _dot(a, b):
    return jnp.dot(a, b, preferred_element_type=_F32)


def _pad2d(x):
    h, w, c = x.shape
    zr = jnp.zeros((1, w, c), x.dtype)
    x = jnp.concatenate([zr, x, zr], axis=0)
    zc = jnp.zeros((h + 2, 1, c), x.dtype)
    return jnp.concatenate([zc, x, zc], axis=1)


def _resblock(x3, w1_ref, b1_ref, w2_ref, b2_ref):
    h = jax.nn.relu(x3)
    hp = _pad2d(h)
    acc = jnp.zeros((_H2 * _H2, _C), _F32)
    for dy in range(3):
        for dx in range(3):
            patch = hp[dy:dy + _H2, dx:dx + _H2, :].reshape(_H2 * _H2, _C)
            acc = acc + _dot(patch, w1_ref[dy, dx])
    h2 = jax.nn.relu(acc + b1_ref[...])
    h3 = _dot(h2, w2_ref[...]) + b2_ref[...]
    return x3 + h3.reshape(_H2, _H2, _C)


def _conv1_body(pat_ref, w_ref, b_ref, out_ref):
    pat = pat_ref[0].reshape(_H1 * _H1, 16 * _XC)
    acc = _dot(pat, w_ref[...]) + b_ref[...]
    out_ref[0] = acc.reshape(_H1, _H1, _C)


def _mega_body(h1p_ref, w2_ref, b2_ref,
               er1w1_ref, er1b1_ref, er1w2_ref, er1b2_ref,
               er2w1_ref, er2b1_ref, er2w2_ref, er2b2_ref,
               codes_t_ref, cn2_ref, codes_ref,
               dr1w1_ref, dr1b1_ref, dr1w2_ref, dr1b2_ref,
               dr2w1_ref, dr2b1_ref, dr2w2_ref, dr2b2_ref,
               dt1w_ref, dt1b_ref,
               zenc_ref, zdec_ref, dph_ref):
    acc = jnp.zeros((_H2 * _H2, _C), _F32)
    for p in range(2):
        for q in range(2):
            hpq = h1p_ref[0, p, q]
            for s in range(2):
                for t in range(2):
                    patch = hpq[s:s + _H2, t:t + _H2, :].reshape(_H2 * _H2, _C)
                    acc = acc + _dot(patch, w2_ref[2 * s + p, 2 * t + q])
    h = (acc + b2_ref[...]).reshape(_H2, _H2, _C)

    h = _resblock(h, er1w1_ref, er1b1_ref, er1w2_ref, er1b2_ref)
    zenc3 = _resblock(h, er2w1_ref, er2b1_ref, er2w2_ref, er2b2_ref)
    zenc_ref[0] = zenc3

    flat = zenc3.reshape(_H2 * _H2, _C)
    rows = (_H2 * _H2) // _VQ_CHUNKS
    zparts = []
    for c in range(_VQ_CHUNKS):
        fc = flat[c * rows:(c + 1) * rows, :]
        scores = _dot(fc, codes_t_ref[...])
        zn = jnp.sum(fc * fc, axis=1, keepdims=True)
        d = zn - 2.0 * scores + cn2_ref[...]
        dmin = jnp.min(d, axis=1, keepdims=True)
        iota = jax.lax.broadcasted_iota(jnp.int32, (rows, _K), 1)
        idx = jnp.min(jnp.where(d == dmin, iota, _K), axis=1,
                      keepdims=True)
        onehot = (iota == idx).astype(_F32)
        zparts.append(_dot(onehot, codes_ref[...]))
    zdec3 = jnp.concatenate(zparts, axis=0).reshape(_H2, _H2, _C)
    zdec_ref[0] = zdec3

    g = _resblock(zdec3, dr1w1_ref, dr1b1_ref, dr1w2_ref, dr1b2_ref)
    g = _resblock(g, dr2w1_ref, dr2b1_ref, dr2w2_ref, dr2b2_ref)

    gp = _pad2d(g)
    phases = []
    for p in range(2):
        for q in range(2):
            acc = jnp.zeros((_H2 * _H2, _C), _F32)
            for sy, ky in _TCONV_TAPS[p]:
                for sx, kx in _TCONV_TAPS[q]:
                    patch = gp[sy:sy + _H2, sx:sx + _H2, :].reshape(_H2 * _H2, _C)
                    acc = acc + _dot(patch, dt1w_ref[ky, kx])
            phases.append(acc + dt1b_ref[...])
    dph_ref[0] = jnp.concatenate(phases, axis=1).reshape(_H2, _H2, 4 * _C)


def _tconv2_body(gp_ref, w_ref, b_ref, xph_ref):
    rc = 14
    for r0 in range(0, _H1, rc):
        phases = []
        for p in range(2):
            for q in range(2):
                acc = jnp.zeros((rc * _H1, _XC), _F32)
                for sy, ky in _TCONV_TAPS[p]:
                    for sx, kx in _TCONV_TAPS[q]:
                        patch = gp_ref[0, r0 + sy:r0 + sy + rc,
                                       sx:sx + _H1, :].reshape(rc * _H1, _C)
                        acc = acc + _dot(patch, w_ref[ky, kx])
                phases.append(acc + b_ref[...])
        xph_ref[0, r0:r0 + rc] = (jnp.concatenate(phases, axis=1)
                                  .reshape(rc, _H1, 4 * _XC))


def _batch_spec(shape):
    n = len(shape)
    return pl.BlockSpec((1,) + tuple(shape[1:]),
                        lambda b: (b,) + (0,) * (n - 1))


def _bcast_spec(shape):
    n = len(shape)
    return pl.BlockSpec(tuple(shape), lambda b: (0,) * n)


def _phase_split(x):
    b, h, w, c = x.shape
    x = x.reshape(b, h // 2, 2, w // 2, 2, c)
    return x.transpose(0, 2, 4, 1, 3, 5)


def kernel(x, codes, ew1, eb1, ew2, eb2, er1w1, er1b1, er1w2, er1b2,
           er2w1, er2b1, er2w2, er2b2, dr1w1, dr1b1, dr1w2, dr1b2,
           dr2w1, dr2b1, dr2w2, dr2b2, dt1w, dt1b, dt2w, dt2b):
    f32 = _F32

    w48 = jnp.concatenate(
        [ew1[:, :, 2 * s + p, 2 * t + q].T for (p, q, s, t) in _C1_TAPS],
        axis=0)
    w2 = ew2.transpose(2, 3, 1, 0)
    def res_w(wa, wb):
        return wa.transpose(2, 3, 1, 0), wb[:, :, 0, 0].T
    er1w1m, er1w2m = res_w(er1w1, er1w2)
    er2w1m, er2w2m = res_w(er2w1, er2w2)
    dr1w1m, dr1w2m = res_w(dr1w1, dr1w2)
    dr2w1m, dr2w2m = res_w(dr2w1, dr2w2)
    dt1wm = dt1w.transpose(2, 3, 0, 1)
    dt2wm = dt2w.transpose(2, 3, 0, 1)
    b_ = lambda v: v.reshape(1, -1)
    codes_t = codes.T
    cn2 = jnp.sum(codes * codes, axis=1).reshape(1, _K)

    xh = x.transpose(0, 2, 3, 1)
    xh = jnp.pad(xh, ((0, 0), (1, 1), (1, 1), (0, 0)))
    xp = _phase_split(xh)
    pat = jnp.concatenate(
        [xp[:, p, q, s:s + _H1, t:t + _H1, :] for (p, q, s, t) in _C1_TAPS],
        axis=3)

    h1 = pl.pallas_call(
        _conv1_body,
        grid=(_B,),
        in_specs=[_batch_spec(pat.shape), _bcast_spec(w48.shape),
                  _bcast_spec((1, _C))],
        out_specs=_batch_spec((_B, _H1, _H1, _C)),
        out_shape=jax.ShapeDtypeStruct((_B, _H1, _H1, _C), f32),
    )(pat, w48, b_(eb1))

    h1p = _phase_split(jnp.pad(h1, ((0, 0), (1, 1), (1, 1), (0, 0))))

    mega_ins = [h1p, w2, b_(eb2),
                er1w1m, b_(er1b1), er1w2m, b_(er1b2),
                er2w1m, b_(er2b1), er2w2m, b_(er2b2),
                codes_t, cn2, codes,
                dr1w1m, b_(dr1b1), dr1w2m, b_(dr1b2),
                dr2w1m, b_(dr2b1), dr2w2m, b_(dr2b2),
                dt1wm, b_(dt1b)]
    in_specs = [_batch_spec(h1p.shape)] + [_bcast_spec(a.shape) for a in mega_ins[1:]]
    zenc, zdec, dph = pl.pallas_call(
        _mega_body,
        grid=(_B,),
        in_specs=in_specs,
        out_specs=[_batch_spec((_B, _H2, _H2, _C)),
                   _batch_spec((_B, _H2, _H2, _C)),
                   _batch_spec((_B, _H2, _H2, 4 * _C))],
        out_shape=[jax.ShapeDtypeStruct((_B, _H2, _H2, _C), f32),
                   jax.ShapeDtypeStruct((_B, _H2, _H2, _C), f32),
                   jax.ShapeDtypeStruct((_B, _H2, _H2, 4 * _C), f32)],
    )(*mega_ins)

    g1 = (dph.reshape(_B, _H2, _H2, 2, 2, _C)
          .transpose(0, 1, 3, 2, 4, 5)
          .reshape(_B, _H1, _H1, _C))
    gp = jnp.pad(g1, ((0, 0), (1, 1), (1, 1), (0, 0)))

    xph = pl.pallas_call(
        _tconv2_body,
        grid=(_B,),
        in_specs=[_batch_spec(gp.shape), _bcast_spec(dt2wm.shape),
                  _bcast_spec((1, _XC))],
        out_specs=_batch_spec((_B, _H1, _H1, 4 * _XC)),
        out_shape=jax.ShapeDtypeStruct((_B, _H1, _H1, 4 * _XC), f32),
    )(gp, dt2wm, b_(dt2b))

    xhat = (xph.reshape(_B, _H1, _H1, 2, 2, _XC)
            .transpose(0, 5, 1, 3, 2, 4)
            .reshape(_B, _XC, _HW, _HW))
    zenc_out = zenc.transpose(0, 3, 1, 2)
    zdec_out = zdec.transpose(0, 3, 1, 2)
    return (xhat, zenc_out, zdec_out)
